# Initial kernel scaffold; baseline (speedup 1.0000x reference)
import jax
import jax.numpy as jnp
from jax.experimental import pallas as pl


def kernel(x, nodes, nbr_l1_self, nbr_l2, nbr_l1_of_l2, W0_w, W0_b, W1_w, W1_b):
    raise NotImplementedError("write your pallas kernel here")


# speedup vs baseline: 3.2752x; 3.2752x over previous
import functools

import jax
import jax.numpy as jnp
from jax import lax
from jax.experimental import pallas as pl
from jax.experimental.pallas import tpu as pltpu
from jax.experimental.pallas import tpu_sc as plsc

D = 128
B = 2048
S = 16

NC = 2
NS = 16
LANES = 16
NW = NC * NS

CH = 128
GPC = CH // S

BIG_CHUNKS = (B * S * S) // NW // CH
L2_CHUNKS = (B * S) // NW // CH
SELF_CHUNKS = (B * S) // NW // CH
TGT_ROWS = B // CH


def _sc_gather(x, idx_l1l2, idx_l2, idx_l1self, idx_nodes):
  mesh = plsc.VectorSubcoreMesh(core_axis_name="c", subcore_axis_name="s",
                                num_cores=NC, num_subcores=NS)
  out_type = (
      jax.ShapeDtypeStruct((B * S, D), jnp.float32),
      jax.ShapeDtypeStruct((B * S, D), jnp.float32),
      jax.ShapeDtypeStruct((B, D), jnp.float32),
      jax.ShapeDtypeStruct((B, D), jnp.float32),
  )
  scratch = [
      pltpu.VMEM((BIG_CHUNKS, CH), jnp.int32),
      pltpu.VMEM((CH, D), jnp.float32),
      pltpu.VMEM((GPC, D), jnp.float32),
      pltpu.SemaphoreType.DMA,
  ]

  @functools.partial(pl.kernel, out_type=out_type, mesh=mesh,
                     scratch_types=scratch)
  def k(x_h, il1l2_h, il2_h, ilself_h, inodes_h,
        o_sum_l1l2, o_l2, o_sum_self, o_tgt,
        idx_v, rows_v, acc_v, sem):
    wid = lax.axis_index("s") * NC + lax.axis_index("c")

    def accumulate_chunk(out_ref, out_base, c):
      for g in range(GPC):
        for d in range(D // LANES):
          a = rows_v[g * S, pl.ds(d * LANES, LANES)]
          for n in range(1, S):
            a = a + rows_v[g * S + n, pl.ds(d * LANES, LANES)]
          acc_v[g, pl.ds(d * LANES, LANES)] = a
      pltpu.sync_copy(acc_v, out_ref.at[pl.ds(out_base + c * GPC, GPC), :])

    pltpu.sync_copy(il1l2_h.at[pl.ds(wid * BIG_CHUNKS, BIG_CHUNKS), :], idx_v)

    def body1(c, carry):
      pltpu.async_copy(x_h.at[idx_v.at[c]], rows_v, sem).wait()
      accumulate_chunk(o_sum_l1l2, wid * (BIG_CHUNKS * GPC), c)
      return carry
    lax.fori_loop(0, BIG_CHUNKS, body1, 0)

    pltpu.sync_copy(il2_h.at[pl.ds(wid * L2_CHUNKS, L2_CHUNKS), :],
                    idx_v.at[pl.ds(0, L2_CHUNKS), :])

    def body2(c, carry):
      pltpu.async_copy(x_h.at[idx_v.at[c]], rows_v, sem).wait()
      pltpu.sync_copy(rows_v, o_l2.at[pl.ds(wid * (L2_CHUNKS * CH) + c * CH,
                                            CH), :])
      return carry
    lax.fori_loop(0, L2_CHUNKS, body2, 0)

    pltpu.sync_copy(ilself_h.at[pl.ds(wid * SELF_CHUNKS, SELF_CHUNKS), :],
                    idx_v.at[pl.ds(0, SELF_CHUNKS), :])

    def body3(c, carry):
      pltpu.async_copy(x_h.at[idx_v.at[c]], rows_v, sem).wait()
      accumulate_chunk(o_sum_self, wid * (SELF_CHUNKS * GPC), c)
      return carry
    lax.fori_loop(0, SELF_CHUNKS, body3, 0)

    @pl.when(wid < TGT_ROWS)
    def _():
      pltpu.sync_copy(inodes_h.at[pl.ds(wid, 1), :],
                      idx_v.at[pl.ds(0, 1), :])
      pltpu.async_copy(x_h.at[idx_v.at[0]], rows_v, sem).wait()
      pltpu.sync_copy(rows_v, o_tgt.at[pl.ds(wid * CH, CH), :])

  return k(x, idx_l1l2, idx_l2, idx_l1self, idx_nodes)


def _sage(h_self, h_mean, wa_ref, wb_ref, b_ref):
  t = jnp.dot(h_self, wa_ref[...], preferred_element_type=jnp.float32,
              precision=lax.Precision.HIGHEST)
  t = t + jnp.dot(h_mean, wb_ref[...], preferred_element_type=jnp.float32,
                  precision=lax.Precision.HIGHEST)
  t = jnp.maximum(t + b_ref[...], 0.0)
  n = jnp.sqrt(jnp.sum(t * t, axis=1, keepdims=True))
  return t / jnp.where(n > 0, n, 1.0)


def _tc_layer1_mean(h0_l2, sum_l1l2, W0a, W0b, b0):
  BLK = 512

  def body(h_ref, s_ref, wa_ref, wb_ref, b_ref, o_ref):
    h1 = _sage(h_ref[...], s_ref[...] * (1.0 / S), wa_ref, wb_ref, b_ref)
    o_ref[...] = jnp.mean(h1.reshape(BLK // S, S, D), axis=1)

  return pl.pallas_call(
      body,
      grid=(B * S // BLK,),
      in_specs=[
          pl.BlockSpec((BLK, D), lambda i: (i, 0)),
          pl.BlockSpec((BLK, D), lambda i: (i, 0)),
          pl.BlockSpec((D, D), lambda i: (0, 0)),
          pl.BlockSpec((D, D), lambda i: (0, 0)),
          pl.BlockSpec((1, D), lambda i: (0, 0)),
      ],
      out_specs=pl.BlockSpec((BLK // S, D), lambda i: (i, 0)),
      out_shape=jax.ShapeDtypeStruct((B, D), jnp.float32),
  )(h0_l2, sum_l1l2, W0a, W0b, b0)


def _tc_layer2(h0_tgt, sum_self, h1n_mean, W0a, W0b, b0, W1a, W1b, b1):
  BLK = 512

  def body(t_ref, s_ref, m_ref, w0a, w0b, b0_ref, w1a, w1b, b1_ref, o_ref):
    h1s = _sage(t_ref[...], s_ref[...] * (1.0 / S), w0a, w0b, b0_ref)
    o_ref[...] = _sage(h1s, m_ref[...], w1a, w1b, b1_ref)

  return pl.pallas_call(
      body,
      grid=(B // BLK,),
      in_specs=[
          pl.BlockSpec((BLK, D), lambda i: (i, 0)),
          pl.BlockSpec((BLK, D), lambda i: (i, 0)),
          pl.BlockSpec((BLK, D), lambda i: (i, 0)),
          pl.BlockSpec((D, D), lambda i: (0, 0)),
          pl.BlockSpec((D, D), lambda i: (0, 0)),
          pl.BlockSpec((1, D), lambda i: (0, 0)),
          pl.BlockSpec((D, D), lambda i: (0, 0)),
          pl.BlockSpec((D, D), lambda i: (0, 0)),
          pl.BlockSpec((1, D), lambda i: (0, 0)),
      ],
      out_specs=pl.BlockSpec((BLK, D), lambda i: (i, 0)),
      out_shape=jax.ShapeDtypeStruct((B, D), jnp.float32),
  )(h0_tgt, sum_self, h1n_mean, W0a, W0b, b0, W1a, W1b, b1)


def kernel(x, nodes, nbr_l1_self, nbr_l2, nbr_l1_of_l2, W0_w, W0_b, W1_w,
           W1_b):
  idx_l1l2 = nbr_l1_of_l2.astype(jnp.int32).reshape(B * S * S // CH, CH)
  idx_l2 = nbr_l2.astype(jnp.int32).reshape(B * S // CH, CH)
  idx_l1self = nbr_l1_self.astype(jnp.int32).reshape(B * S // CH, CH)
  idx_nodes = nodes.astype(jnp.int32).reshape(TGT_ROWS, CH)

  sum_l1l2, h0_l2, sum_self, h0_tgt = _sc_gather(
      x, idx_l1l2, idx_l2, idx_l1self, idx_nodes)

  W0a, W0b = W0_w[:D], W0_w[D:]
  W1a, W1b = W1_w[:D], W1_w[D:]
  b0 = W0_b.reshape(1, D)
  b1 = W1_b.reshape(1, D)

  h1n_mean = _tc_layer1_mean(h0_l2, sum_l1l2, W0a, W0b, b0)
  z = _tc_layer2(h0_tgt, sum_self, h1n_mean, W0a, W0b, b0, W1a, W1b, b1)
  return z


# speedup vs baseline: 4.4260x; 1.3514x over previous
import functools

import jax
import jax.numpy as jnp
from jax import lax
from jax.experimental import pallas as pl
from jax.experimental.pallas import tpu as pltpu
from jax.experimental.pallas import tpu_sc as plsc

D = 128
B = 2048
S = 16

NC = 2
NS = 16
LANES = 16
NW = NC * NS

CH = 128
GPC = CH // S

BIG_CHUNKS = (B * S * S) // NW // CH
L2_CHUNKS = (B * S) // NW // CH
SELF_CHUNKS = (B * S) // NW // CH
TGT_ROWS = B // CH


def _sc_gather(x, idx_l1l2, idx_l2, idx_l1self, idx_nodes):
  mesh = plsc.VectorSubcoreMesh(core_axis_name="c", subcore_axis_name="s",
                                num_cores=NC, num_subcores=NS)
  out_type = (
      jax.ShapeDtypeStruct((B * S, D), jnp.float32),
      jax.ShapeDtypeStruct((B * S, D), jnp.float32),
      jax.ShapeDtypeStruct((B, D), jnp.float32),
      jax.ShapeDtypeStruct((B, D), jnp.float32),
  )
  scratch = [
      pltpu.VMEM((BIG_CHUNKS, CH), jnp.int32),
      pltpu.VMEM((CH, D), jnp.float32),
      pltpu.VMEM((CH, D), jnp.float32),
      pltpu.VMEM((2 * GPC, D), jnp.float32),
      pltpu.SemaphoreType.DMA,
      pltpu.SemaphoreType.DMA,
  ]

  @functools.partial(pl.kernel, out_type=out_type, mesh=mesh,
                     scratch_types=scratch)
  def k(x_h, il1l2_h, il2_h, ilself_h, inodes_h,
        o_sum_l1l2, o_l2, o_sum_self, o_tgt,
        idx_v, rows_a, rows_b, acc_v, sem_a, sem_b):
    wid = lax.axis_index("s") * NC + lax.axis_index("c")

    def gather_start(c, rows_v, sem):
      pltpu.async_copy(x_h.at[idx_v.at[c]], rows_v, sem)

    def gather_wait(c, rows_v, sem):
      pltpu.make_async_copy(x_h.at[idx_v.at[c]], rows_v, sem).wait()

    def accumulate(rows_v, acc_base):
      for g in range(GPC):
        for d in range(D // LANES):
          a = rows_v[g * S, pl.ds(d * LANES, LANES)]
          for n in range(1, S):
            a = a + rows_v[g * S + n, pl.ds(d * LANES, LANES)]
          acc_v[acc_base + g, pl.ds(d * LANES, LANES)] = a

    def segsum_stage(n_chunks, out_ref, out_base):
      gather_start(0, rows_a, sem_a)
      gather_start(1, rows_b, sem_b)

      def body(i, carry):
        c0 = 2 * i
        gather_wait(c0, rows_a, sem_a)
        accumulate(rows_a, 0)

        @pl.when(c0 + 2 < n_chunks)
        def _():
          gather_start(c0 + 2, rows_a, sem_a)

        gather_wait(c0 + 1, rows_b, sem_b)
        accumulate(rows_b, GPC)

        @pl.when(c0 + 3 < n_chunks)
        def _():
          gather_start(c0 + 3, rows_b, sem_b)

        pltpu.sync_copy(acc_v,
                        out_ref.at[pl.ds(out_base + i * 2 * GPC, 2 * GPC), :])
        return carry
      lax.fori_loop(0, n_chunks // 2, body, 0)

    pltpu.sync_copy(il1l2_h.at[pl.ds(wid * BIG_CHUNKS, BIG_CHUNKS), :], idx_v)
    segsum_stage(BIG_CHUNKS, o_sum_l1l2, wid * (BIG_CHUNKS * GPC))

    pltpu.sync_copy(il2_h.at[pl.ds(wid * L2_CHUNKS, L2_CHUNKS), :],
                    idx_v.at[pl.ds(0, L2_CHUNKS), :])
    gather_start(0, rows_a, sem_a)
    gather_start(1, rows_b, sem_b)

    def body2(i, carry):
      c0 = 2 * i
      base = wid * (L2_CHUNKS * CH)
      gather_wait(c0, rows_a, sem_a)
      pltpu.sync_copy(rows_a, o_l2.at[pl.ds(base + c0 * CH, CH), :])

      @pl.when(c0 + 2 < L2_CHUNKS)
      def _():
        gather_start(c0 + 2, rows_a, sem_a)

      gather_wait(c0 + 1, rows_b, sem_b)
      pltpu.sync_copy(rows_b, o_l2.at[pl.ds(base + (c0 + 1) * CH, CH), :])

      @pl.when(c0 + 3 < L2_CHUNKS)
      def _():
        gather_start(c0 + 3, rows_b, sem_b)
      return carry
    lax.fori_loop(0, L2_CHUNKS // 2, body2, 0)

    pltpu.sync_copy(ilself_h.at[pl.ds(wid * SELF_CHUNKS, SELF_CHUNKS), :],
                    idx_v.at[pl.ds(0, SELF_CHUNKS), :])
    segsum_stage(SELF_CHUNKS, o_sum_self, wid * (SELF_CHUNKS * GPC))

    @pl.when(wid < TGT_ROWS)
    def _():
      pltpu.sync_copy(inodes_h.at[pl.ds(wid, 1), :],
                      idx_v.at[pl.ds(0, 1), :])
      pltpu.async_copy(x_h.at[idx_v.at[0]], rows_a, sem_a).wait()
      pltpu.sync_copy(rows_a, o_tgt.at[pl.ds(wid * CH, CH), :])

  return k(x, idx_l1l2, idx_l2, idx_l1self, idx_nodes)


def _sage(h_self, h_mean, wa_ref, wb_ref, b_ref):
  t = jnp.dot(h_self, wa_ref[...], preferred_element_type=jnp.float32,
              precision=lax.Precision.HIGHEST)
  t = t + jnp.dot(h_mean, wb_ref[...], preferred_element_type=jnp.float32,
                  precision=lax.Precision.HIGHEST)
  t = jnp.maximum(t + b_ref[...], 0.0)
  n = jnp.sqrt(jnp.sum(t * t, axis=1, keepdims=True))
  return t / jnp.where(n > 0, n, 1.0)


def _tc_layer1_mean(h0_l2, sum_l1l2, W0a, W0b, b0):
  BLK = 512

  def body(h_ref, s_ref, wa_ref, wb_ref, b_ref, o_ref):
    h1 = _sage(h_ref[...], s_ref[...] * (1.0 / S), wa_ref, wb_ref, b_ref)
    o_ref[...] = jnp.mean(h1.reshape(BLK // S, S, D), axis=1)

  return pl.pallas_call(
      body,
      grid=(B * S // BLK,),
      in_specs=[
          pl.BlockSpec((BLK, D), lambda i: (i, 0)),
          pl.BlockSpec((BLK, D), lambda i: (i, 0)),
          pl.BlockSpec((D, D), lambda i: (0, 0)),
          pl.BlockSpec((D, D), lambda i: (0, 0)),
          pl.BlockSpec((1, D), lambda i: (0, 0)),
      ],
      out_specs=pl.BlockSpec((BLK // S, D), lambda i: (i, 0)),
      out_shape=jax.ShapeDtypeStruct((B, D), jnp.float32),
  )(h0_l2, sum_l1l2, W0a, W0b, b0)


def _tc_layer2(h0_tgt, sum_self, h1n_mean, W0a, W0b, b0, W1a, W1b, b1):
  BLK = 512

  def body(t_ref, s_ref, m_ref, w0a, w0b, b0_ref, w1a, w1b, b1_ref, o_ref):
    h1s = _sage(t_ref[...], s_ref[...] * (1.0 / S), w0a, w0b, b0_ref)
    o_ref[...] = _sage(h1s, m_ref[...], w1a, w1b, b1_ref)

  return pl.pallas_call(
      body,
      grid=(B // BLK,),
      in_specs=[
          pl.BlockSpec((BLK, D), lambda i: (i, 0)),
          pl.BlockSpec((BLK, D), lambda i: (i, 0)),
          pl.BlockSpec((BLK, D), lambda i: (i, 0)),
          pl.BlockSpec((D, D), lambda i: (0, 0)),
          pl.BlockSpec((D, D), lambda i: (0, 0)),
          pl.BlockSpec((1, D), lambda i: (0, 0)),
          pl.BlockSpec((D, D), lambda i: (0, 0)),
          pl.BlockSpec((D, D), lambda i: (0, 0)),
          pl.BlockSpec((1, D), lambda i: (0, 0)),
      ],
      out_specs=pl.BlockSpec((BLK, D), lambda i: (i, 0)),
      out_shape=jax.ShapeDtypeStruct((B, D), jnp.float32),
  )(h0_tgt, sum_self, h1n_mean, W0a, W0b, b0, W1a, W1b, b1)


def kernel(x, nodes, nbr_l1_self, nbr_l2, nbr_l1_of_l2, W0_w, W0_b, W1_w,
           W1_b):
  idx_l1l2 = nbr_l1_of_l2.astype(jnp.int32).reshape(B * S * S // CH, CH)
  idx_l2 = nbr_l2.astype(jnp.int32).reshape(B * S // CH, CH)
  idx_l1self = nbr_l1_self.astype(jnp.int32).reshape(B * S // CH, CH)
  idx_nodes = nodes.astype(jnp.int32).reshape(TGT_ROWS, CH)

  sum_l1l2, h0_l2, sum_self, h0_tgt = _sc_gather(
      x, idx_l1l2, idx_l2, idx_l1self, idx_nodes)

  W0a, W0b = W0_w[:D], W0_w[D:]
  W1a, W1b = W1_w[:D], W1_w[D:]
  b0 = W0_b.reshape(1, D)
  b1 = W1_b.reshape(1, D)

  h1n_mean = _tc_layer1_mean(h0_l2, sum_l1l2, W0a, W0b, b0)
  z = _tc_layer2(h0_tgt, sum_self, h1n_mean, W0a, W0b, b0, W1a, W1b, b1)
  return z
